# baseline (speedup 1.0000x reference)
import functools

import jax
import jax.numpy as jnp
from jax import lax
from jax.experimental import pallas as pl
from jax.experimental.pallas import tpu as pltpu
from jax.experimental.pallas import tpu_sc as plsc

D = 768
L = 16
DJ = D // L
NC = 2
NS = 16
NW = NC * NS
K = 64


def _rsqrt16(x):
    i = plsc.bitcast(x, jnp.int32)
    y = plsc.bitcast(jnp.int32(0x5F3759DF) - (i >> 1), jnp.float32)
    half = x * 0.5
    for _ in range(3):
        y = y * (1.5 - half * y * y)
    return y


def _body(tid_hbm, pid_hbm, tok_hbm, pos_hbm, gam_hbm, bet_hbm, out_hbm,
          tidx, pidx, rows, prows, gam_v, bet_v, sem, sem2):
    wid = lax.axis_index("s") * NC + lax.axis_index("c")
    n_tok = tid_hbm.shape[0]
    per_w = n_tok // NW
    n_chunks = per_w // K
    base = wid * per_w

    pltpu.sync_copy(gam_hbm, gam_v)
    pltpu.sync_copy(bet_hbm, bet_v)

    zeros = jnp.zeros((L,), jnp.float32)

    def chunk_body(c, _):
        off = base + c * K
        pltpu.sync_copy(tid_hbm.at[pl.ds(off, K)], tidx)
        pltpu.sync_copy(pid_hbm.at[pl.ds(off, K)], pidx)
        ct = pltpu.async_copy(tok_hbm.at[tidx], rows, sem)
        cp = pltpu.async_copy(pos_hbm.at[pidx], prows, sem2)
        ct.wait()
        cp.wait()

        def token_body(t, _):
            def p1(j, carry):
                s, q = carry
                sl = pl.ds(j * L, L)
                v = rows[t, sl] + prows[t, sl]
                rows[t, sl] = v
                return s + v, q + v * v

            s, q = lax.fori_loop(0, DJ, p1, (zeros, zeros))
            tot = jnp.sum(s)
            tsq = jnp.sum(q)
            mean = tot * (1.0 / D)
            var = tsq * (1.0 / D) - mean * mean
            rstd = _rsqrt16(jnp.full((L,), var + 1e-5, jnp.float32))
            mean_v = jnp.full((L,), mean, jnp.float32)

            def p2(j, _):
                sl = pl.ds(j * L, L)
                v = rows[t, sl]
                rows[t, sl] = (v - mean_v) * rstd * gam_v[sl] + bet_v[sl]
                return 0

            lax.fori_loop(0, DJ, p2, 0)
            return 0

        lax.fori_loop(0, K, token_body, 0)
        pltpu.sync_copy(rows, out_hbm.at[pl.ds(off, K)])
        return 0

    lax.fori_loop(0, n_chunks, chunk_body, 0)


@jax.jit
def _emb(tid, pid, tok_table, pos_table, ln_gamma, ln_beta):
    n_tok = tid.shape[0]
    mesh = plsc.VectorSubcoreMesh(
        core_axis_name="c", subcore_axis_name="s",
        num_cores=NC, num_subcores=NS)
    return pl.kernel(
        _body,
        out_type=jax.ShapeDtypeStruct((n_tok, D), jnp.float32),
        mesh=mesh,
        compiler_params=pltpu.CompilerParams(needs_layout_passes=False),
        scratch_types=[
            pltpu.VMEM((K,), jnp.int32),
            pltpu.VMEM((K,), jnp.int32),
            pltpu.VMEM((K, D), jnp.float32),
            pltpu.VMEM((K, D), jnp.float32),
            pltpu.VMEM((D,), jnp.float32),
            pltpu.VMEM((D,), jnp.float32),
            pltpu.SemaphoreType.DMA,
            pltpu.SemaphoreType.DMA,
        ],
    )(tid, pid, tok_table, pos_table, ln_gamma, ln_beta)


def kernel(token_ids, position_ids, tok_table, pos_table, ln_gamma, ln_beta):
    b, s = token_ids.shape
    tid = token_ids.reshape(-1).astype(jnp.int32)
    pid = position_ids.reshape(-1).astype(jnp.int32)
    out = _emb(tid, pid, tok_table, pos_table, ln_gamma, ln_beta)
    return out.reshape(b, s, D)


# speedup vs baseline: 1.8807x; 1.8807x over previous
import jax
import jax.numpy as jnp
from jax import lax
from jax.experimental import pallas as pl
from jax.experimental.pallas import tpu as pltpu
from jax.experimental.pallas import tpu_sc as plsc

D = 768
L = 16
DJ = D // L
NC = 2
NS = 16
NW = NC * NS
K = 16


def _rsqrt16(x):
    i = plsc.bitcast(x, jnp.int32)
    y = plsc.bitcast(jnp.int32(0x5F3759DF) - (i >> 1), jnp.float32)
    half = x * 0.5
    for _ in range(3):
        y = y * (1.5 - half * y * y)
    return y


def _body(tid_hbm, pid_hbm, tok_hbm, pos_hbm, gam_hbm, bet_hbm, out_hbm,
          tidx, pidx, rows0, rows1, prows0, prows1, obuf0, obuf1,
          gam_v, bet_v, gsem0, gsem1, psem0, psem1, osem0, osem1):
    rows = (rows0, rows1)
    prows = (prows0, prows1)
    obuf = (obuf0, obuf1)
    gsem = (gsem0, gsem1)
    psem = (psem0, psem1)
    osem = (osem0, osem1)

    wid = lax.axis_index("s") * NC + lax.axis_index("c")
    n_tok = tid_hbm.shape[0]
    per_w = n_tok // NW
    n_chunks = per_w // K
    base = wid * per_w

    pltpu.sync_copy(tid_hbm.at[pl.ds(base, per_w)], tidx)
    pltpu.sync_copy(pid_hbm.at[pl.ds(base, per_w)], pidx)
    pltpu.sync_copy(gam_hbm, gam_v)
    pltpu.sync_copy(bet_hbm, bet_v)

    def fire_gathers(c, b):
        pltpu.async_copy(tok_hbm.at[tidx.at[pl.ds(c * K, K)]], rows[b],
                         gsem[b])
        pltpu.async_copy(pos_hbm.at[pidx.at[pl.ds(c * K, K)]], prows[b],
                         psem[b])

    fire_gathers(0, 0)
    fire_gathers(1, 1)

    def compute_chunk(b):
        def token_body(t, _):
            vs = []
            s = jnp.zeros((L,), jnp.float32)
            q = jnp.zeros((L,), jnp.float32)
            for j in range(DJ):
                sl = pl.ds(j * L, L)
                v = rows[b][t, sl] + prows[b][t, sl]
                vs.append(v)
                s = s + v
                q = q + v * v
            mean = jnp.sum(s) * (1.0 / D)
            var = jnp.sum(q) * (1.0 / D) - mean * mean
            rstd = _rsqrt16(jnp.full((L,), var + 1e-5, jnp.float32))
            mean_v = jnp.full((L,), mean, jnp.float32)
            for j in range(DJ):
                sl = pl.ds(j * L, L)
                obuf[b][t, sl] = (vs[j] - mean_v) * rstd * gam_v[sl] \
                    + bet_v[sl]
            return 0

        lax.fori_loop(0, K, token_body, 0)

    def loop_body(i, _):
        for b in (0, 1):
            c = 2 * i + b
            pltpu.make_async_copy(
                tok_hbm.at[tidx.at[pl.ds(c * K, K)]], rows[b],
                gsem[b]).wait()
            pltpu.make_async_copy(
                pos_hbm.at[pidx.at[pl.ds(c * K, K)]], prows[b],
                psem[b]).wait()

            @pl.when(c >= 2)
            def _():
                pltpu.make_async_copy(
                    obuf[b], out_hbm.at[pl.ds(base, K)], osem[b]).wait()

            compute_chunk(b)

            pltpu.async_copy(obuf[b], out_hbm.at[pl.ds(base + c * K, K)],
                             osem[b])

            @pl.when(c + 2 < n_chunks)
            def _():
                fire_gathers(c + 2, b)
        return 0

    lax.fori_loop(0, n_chunks // 2, loop_body, 0)

    for b in (0, 1):
        pltpu.make_async_copy(
            obuf[b], out_hbm.at[pl.ds(base, K)], osem[b]).wait()


@jax.jit
def _emb(tid, pid, tok_table, pos_table, ln_gamma, ln_beta):
    n_tok = tid.shape[0]
    mesh = plsc.VectorSubcoreMesh(
        core_axis_name="c", subcore_axis_name="s",
        num_cores=NC, num_subcores=NS)
    per_w = n_tok // NW
    return pl.kernel(
        _body,
        out_type=jax.ShapeDtypeStruct((n_tok, D), jnp.float32),
        mesh=mesh,
        compiler_params=pltpu.CompilerParams(needs_layout_passes=False),
        scratch_types=[
            pltpu.VMEM((per_w,), jnp.int32),
            pltpu.VMEM((per_w,), jnp.int32),
            pltpu.VMEM((K, D), jnp.float32),
            pltpu.VMEM((K, D), jnp.float32),
            pltpu.VMEM((K, D), jnp.float32),
            pltpu.VMEM((K, D), jnp.float32),
            pltpu.VMEM((K, D), jnp.float32),
            pltpu.VMEM((K, D), jnp.float32),
            pltpu.VMEM((D,), jnp.float32),
            pltpu.VMEM((D,), jnp.float32),
            pltpu.SemaphoreType.DMA,
            pltpu.SemaphoreType.DMA,
            pltpu.SemaphoreType.DMA,
            pltpu.SemaphoreType.DMA,
            pltpu.SemaphoreType.DMA,
            pltpu.SemaphoreType.DMA,
        ],
    )(tid, pid, tok_table, pos_table, ln_gamma, ln_beta)


def kernel(token_ids, position_ids, tok_table, pos_table, ln_gamma, ln_beta):
    b, s = token_ids.shape
    tid = token_ids.reshape(-1).astype(jnp.int32)
    pid = position_ids.reshape(-1).astype(jnp.int32)
    out = _emb(tid, pid, tok_table, pos_table, ln_gamma, ln_beta)
    return out.reshape(b, s, D)


# speedup vs baseline: 2.9944x; 1.5922x over previous
import jax
import jax.numpy as jnp
from jax import lax
from jax.experimental import pallas as pl
from jax.experimental.pallas import tpu as pltpu
from jax.experimental.pallas import tpu_sc as plsc

D = 768
L = 16
DJ = D // L
NC = 2
NS = 16
NW = NC * NS
K = 16


def _rsqrt16(x):
    i = plsc.bitcast(x, jnp.int32)
    y = plsc.bitcast(jnp.int32(0x5F3759DF) - (i >> 1), jnp.float32)
    half = x * 0.5
    for _ in range(3):
        y = y * (1.5 - half * y * y)
    return y


def _body(tid_hbm, pid_hbm, tok_hbm, pos_hbm, gam_hbm, bet_hbm, out_hbm,
          tidx, pidx, rows0, rows1, prows0, prows1, obuf0, obuf1,
          stats0, stats1, gam_v, bet_v,
          gsem0, gsem1, psem0, psem1, osem0, osem1):
    rows = (rows0, rows1)
    prows = (prows0, prows1)
    obuf = (obuf0, obuf1)
    stats = (stats0, stats1)
    gsem = (gsem0, gsem1)
    psem = (psem0, psem1)
    osem = (osem0, osem1)

    wid = lax.axis_index("s") * NC + lax.axis_index("c")
    n_tok = tid_hbm.shape[0]
    per_w = n_tok // NW
    n_chunks = per_w // K
    base = wid * per_w

    pltpu.sync_copy(tid_hbm.at[pl.ds(base, per_w)], tidx)
    pltpu.sync_copy(pid_hbm.at[pl.ds(base, per_w)], pidx)
    pltpu.sync_copy(gam_hbm, gam_v)
    pltpu.sync_copy(bet_hbm, bet_v)

    def fire_gathers(c, b):
        pltpu.async_copy(tok_hbm.at[tidx.at[pl.ds(c * K, K)]], rows[b],
                         gsem[b])
        pltpu.async_copy(pos_hbm.at[pidx.at[pl.ds(c * K, K)]], prows[b],
                         psem[b])

    fire_gathers(0, 0)
    fire_gathers(1, 1)

    def compute_chunk(b):
        def token_stats(t, _):
            ss = [jnp.zeros((L,), jnp.float32) for _ in range(4)]
            qq = [jnp.zeros((L,), jnp.float32) for _ in range(4)]
            for j in range(DJ):
                sl = pl.ds(j * L, L)
                v = rows[b][t, sl] + prows[b][t, sl]
                rows[b][t, sl] = v
                ss[j % 4] = ss[j % 4] + v
                qq[j % 4] = qq[j % 4] + v * v
            s = (ss[0] + ss[1]) + (ss[2] + ss[3])
            q = (qq[0] + qq[1]) + (qq[2] + qq[3])
            mean = jnp.sum(s) * (1.0 / D)
            var = jnp.sum(q) * (1.0 / D) - mean * mean
            rstd = _rsqrt16(jnp.full((L,), var + 1e-5, jnp.float32))
            stats[b][0, t, :] = rstd
            stats[b][1, t, :] = jnp.full((L,), mean, jnp.float32) * rstd
            return 0

        lax.fori_loop(0, K, token_stats, 0)

        a_regs = [stats[b][0, t, :] for t in range(K)]
        c_regs = [stats[b][1, t, :] for t in range(K)]

        def jbody(j, _):
            sl = pl.ds(j * L, L)
            g = gam_v[sl]
            bb = bet_v[sl]
            for t in range(K):
                v = rows[b][t, sl]
                obuf[b][t, sl] = (v * a_regs[t] - c_regs[t]) * g + bb
            return 0

        lax.fori_loop(0, DJ, jbody, 0)

    def loop_body(i, _):
        for b in (0, 1):
            c = 2 * i + b
            pltpu.make_async_copy(
                tok_hbm.at[tidx.at[pl.ds(c * K, K)]], rows[b],
                gsem[b]).wait()
            pltpu.make_async_copy(
                pos_hbm.at[pidx.at[pl.ds(c * K, K)]], prows[b],
                psem[b]).wait()

            @pl.when(c >= 2)
            def _():
                pltpu.make_async_copy(
                    obuf[b], out_hbm.at[pl.ds(base, K)], osem[b]).wait()

            compute_chunk(b)

            pltpu.async_copy(obuf[b], out_hbm.at[pl.ds(base + c * K, K)],
                             osem[b])

            @pl.when(c + 2 < n_chunks)
            def _():
                fire_gathers(c + 2, b)
        return 0

    lax.fori_loop(0, n_chunks // 2, loop_body, 0)

    for b in (0, 1):
        pltpu.make_async_copy(
            obuf[b], out_hbm.at[pl.ds(base, K)], osem[b]).wait()


@jax.jit
def _emb(tid, pid, tok_table, pos_table, ln_gamma, ln_beta):
    n_tok = tid.shape[0]
    mesh = plsc.VectorSubcoreMesh(
        core_axis_name="c", subcore_axis_name="s",
        num_cores=NC, num_subcores=NS)
    per_w = n_tok // NW
    return pl.kernel(
        _body,
        out_type=jax.ShapeDtypeStruct((n_tok, D), jnp.float32),
        mesh=mesh,
        compiler_params=pltpu.CompilerParams(needs_layout_passes=False),
        scratch_types=[
            pltpu.VMEM((per_w,), jnp.int32),
            pltpu.VMEM((per_w,), jnp.int32),
            pltpu.VMEM((K, D), jnp.float32),
            pltpu.VMEM((K, D), jnp.float32),
            pltpu.VMEM((K, D), jnp.float32),
            pltpu.VMEM((K, D), jnp.float32),
            pltpu.VMEM((K, D), jnp.float32),
            pltpu.VMEM((K, D), jnp.float32),
            pltpu.VMEM((2, K, L), jnp.float32),
            pltpu.VMEM((2, K, L), jnp.float32),
            pltpu.VMEM((D,), jnp.float32),
            pltpu.VMEM((D,), jnp.float32),
            pltpu.SemaphoreType.DMA,
            pltpu.SemaphoreType.DMA,
            pltpu.SemaphoreType.DMA,
            pltpu.SemaphoreType.DMA,
            pltpu.SemaphoreType.DMA,
            pltpu.SemaphoreType.DMA,
        ],
    )(tid, pid, tok_table, pos_table, ln_gamma, ln_beta)


def kernel(token_ids, position_ids, tok_table, pos_table, ln_gamma, ln_beta):
    b, s = token_ids.shape
    tid = token_ids.reshape(-1).astype(jnp.int32)
    pid = position_ids.reshape(-1).astype(jnp.int32)
    out = _emb(tid, pid, tok_table, pos_table, ln_gamma, ln_beta)
    return out.reshape(b, s, D)
